# baseline (speedup 1.0000x reference)
import jax
import jax.numpy as jnp
from jax.experimental import pallas as pl
from jax.experimental.pallas import tpu as pltpu

B = 1024
I = 16
F = 512
K = 512
N_LAYERS = 4
BB = 128
OCT = 8


def _prep_weights(a_ref, b_ref, a_sc, b_sc):
    a = a_ref[...]
    inv = jax.lax.rsqrt(jnp.sum(a * a, axis=1, keepdims=True))
    a_sc[...] = (a * inv).astype(jnp.bfloat16)
    b_sc[...] = b_ref[...].astype(jnp.bfloat16)


def _layer_body(x_ref, a_ref, b_ref, inner_ref, xnew_ref, xflat_ref,
                a_sc, b_sc):
    bb = pl.program_id(1)
    pl.when(bb == 0)(lambda: _prep_weights(a_ref, b_ref, a_sc, b_sc))
    for i in range(OCT):
        xi = x_ref[:, i * F:(i + 1) * F]
        inner = jnp.dot(xi, a_sc[i], preferred_element_type=jnp.float32)
        inner_ref[:, i, :] = inner
        xn = jnp.dot(inner.astype(jnp.bfloat16), b_sc[i],
                     preferred_element_type=jnp.float32)
        xnew_ref[:, i, :] = xn
        xflat_ref[:, i * F:(i + 1) * F] = xn.astype(jnp.bfloat16)


def _layer_body_last(x_ref, a_ref, b_ref, inner_ref, xnew_ref,
                     a_sc, b_sc):
    bb = pl.program_id(1)
    pl.when(bb == 0)(lambda: _prep_weights(a_ref, b_ref, a_sc, b_sc))
    for i in range(OCT):
        xi = x_ref[:, i * F:(i + 1) * F]
        inner = jnp.dot(xi, a_sc[i], preferred_element_type=jnp.float32)
        inner_ref[:, i, :] = inner
        xn = jnp.dot(inner.astype(jnp.bfloat16), b_sc[i],
                     preferred_element_type=jnp.float32)
        xnew_ref[:, i, :] = xn


def _layer(x_flat, A, Bw, last):
    flat_spec = pl.BlockSpec((BB, OCT * F), lambda o, bb: (bb, o))
    can_spec = pl.BlockSpec((BB, OCT, F), lambda o, bb: (bb, o, 0))
    w_spec = pl.BlockSpec((OCT, F, K), lambda o, bb: (o, 0, 0))
    can_shape = jax.ShapeDtypeStruct((B, I, F), jnp.float32)
    flat_shape = jax.ShapeDtypeStruct((B, I * F), jnp.bfloat16)
    out_specs = [can_spec, can_spec] + ([] if last else [flat_spec])
    out_shape = [can_shape, can_shape] + ([] if last else [flat_shape])
    return pl.pallas_call(
        _layer_body_last if last else _layer_body,
        grid=(I // OCT, B // BB),
        in_specs=[flat_spec, w_spec, w_spec],
        out_specs=out_specs,
        out_shape=out_shape,
        scratch_shapes=[
            pltpu.VMEM((OCT, F, K), jnp.bfloat16),
            pltpu.VMEM((OCT, F, K), jnp.bfloat16),
        ],
        compiler_params=pltpu.CompilerParams(
            dimension_semantics=("arbitrary", "arbitrary"),
            vmem_limit_bytes=67108864,
            internal_scratch_in_bytes=262144,
        ),
    )(x_flat, A, Bw)


def kernel(x, A0, A1, A2, A3, B0, B1, B2, B3):
    x_flat = x.reshape(B, I * F).astype(jnp.bfloat16)
    n0, l0, f0 = _layer(x_flat, A0, B0, last=False)
    n1, l1, f1 = _layer(f0, A1, B1, last=False)
    n2, l2, f2 = _layer(f1, A2, B2, last=False)
    n3, l3 = _layer(f2, A3, B3, last=True)
    return (l3, l0, l1, l2, l3, n0, n1, n2, n3)


# speedup vs baseline: 1.0630x; 1.0630x over previous
import jax
import jax.numpy as jnp
from jax.experimental import pallas as pl
from jax.experimental.pallas import tpu as pltpu

B = 1024
I = 16
F = 512
K = 512
N_LAYERS = 4
BB = 256
BB_LAST = 128
OCT = 8


def _prep_weights(a_ref, a_sc):
    a = a_ref[...]
    inv = jax.lax.rsqrt(jnp.sum(a * a, axis=1, keepdims=True))
    a_sc[...] = (a * inv).astype(jnp.bfloat16)


def _layer_body(x_ref, a_ref, b_ref, inner_ref, xnew_ref, xflat_ref,
                a_sc):
    bb = pl.program_id(1)
    pl.when(bb == 0)(lambda: _prep_weights(a_ref, a_sc))
    for i in range(OCT):
        xi = x_ref[:, i * F:(i + 1) * F]
        inner = jnp.dot(xi, a_sc[i], preferred_element_type=jnp.float32)
        inner_ref[:, i, :] = inner
        xn = jnp.dot(inner.astype(jnp.bfloat16),
                     b_ref[i].astype(jnp.bfloat16),
                     preferred_element_type=jnp.float32)
        xnew_ref[:, i, :] = xn
        xflat_ref[:, i * F:(i + 1) * F] = xn.astype(jnp.bfloat16)


def _layer_body_last(x_ref, a_ref, b_ref, inner_ref, xnew_ref, xdup_ref,
                     a_sc):
    bb = pl.program_id(1)
    pl.when(bb == 0)(lambda: _prep_weights(a_ref, a_sc))
    for i in range(OCT):
        xi = x_ref[:, i * F:(i + 1) * F]
        inner = jnp.dot(xi, a_sc[i], preferred_element_type=jnp.float32)
        inner_ref[:, i, :] = inner
        xn = jnp.dot(inner.astype(jnp.bfloat16),
                     b_ref[i].astype(jnp.bfloat16),
                     preferred_element_type=jnp.float32)
        xnew_ref[:, i, :] = xn
        xdup_ref[:, i, :] = xn


def _layer(x_flat, A, Bw, last):
    blk = BB_LAST if last else BB
    flat_spec = pl.BlockSpec((blk, OCT * F), lambda o, bb: (bb, o))
    can_spec = pl.BlockSpec((blk, OCT, F), lambda o, bb: (bb, o, 0))
    w_spec = pl.BlockSpec((OCT, F, K), lambda o, bb: (o, 0, 0))
    can_shape = jax.ShapeDtypeStruct((B, I, F), jnp.float32)
    flat_shape = jax.ShapeDtypeStruct((B, I * F), jnp.bfloat16)
    out_specs = [can_spec, can_spec, can_spec if last else flat_spec]
    out_shape = [can_shape, can_shape, can_shape if last else flat_shape]
    return pl.pallas_call(
        _layer_body_last if last else _layer_body,
        grid=(I // OCT, B // blk),
        in_specs=[flat_spec, w_spec, w_spec],
        out_specs=out_specs,
        out_shape=out_shape,
        scratch_shapes=[
            pltpu.VMEM((OCT, F, K), jnp.bfloat16),
        ],
        compiler_params=pltpu.CompilerParams(
            dimension_semantics=("arbitrary", "arbitrary"),
            vmem_limit_bytes=67108864,
            internal_scratch_in_bytes=262144,
        ),
    )(x_flat, A, Bw)


def kernel(x, A0, A1, A2, A3, B0, B1, B2, B3):
    x_flat = x.reshape(B, I * F).astype(jnp.bfloat16)
    n0, l0, f0 = _layer(x_flat, A0, B0, last=False)
    n1, l1, f1 = _layer(f0, A1, B1, last=False)
    n2, l2, f2 = _layer(f1, A2, B2, last=False)
    n3, l3, xfin = _layer(f2, A3, B3, last=True)
    return (xfin, l0, l1, l2, l3, n0, n1, n2, n3)
